# baseline (speedup 1.0000x reference)
import functools

import jax
import jax.numpy as jnp
from jax import lax
from jax.experimental import pallas as pl
from jax.experimental.pallas import tpu as pltpu
from jax.experimental.pallas import tpu_sc as plsc

NC = 2
NS = 16
LANES = 16


def _mlp1(x, W1, b1, *, grid_n=2):
    n, d_in = x.shape
    d_hid = W1.shape[1]
    blk = n // grid_n

    def body(x_ref, w_ref, b_ref, o_ref):
        h = jnp.dot(x_ref[...], w_ref[...], preferred_element_type=jnp.float32)
        o_ref[...] = jax.nn.sigmoid(jnp.tanh(h + b_ref[...]))

    return pl.pallas_call(
        body,
        grid=(grid_n,),
        in_specs=[
            pl.BlockSpec((blk, d_in), lambda i: (i, 0)),
            pl.BlockSpec((d_in, d_hid), lambda i: (0, 0)),
            pl.BlockSpec((1, d_hid), lambda i: (0, 0)),
        ],
        out_specs=pl.BlockSpec((blk, d_hid), lambda i: (i, 0)),
        out_shape=jax.ShapeDtypeStruct((n, d_hid), jnp.float32),
    )(x, W1, b1.reshape(1, d_hid))


K = 80
G = 5


def _sc_aggregate(hidden2, idx3):
    n2, dh = hidden2.shape
    n = n2 // NC
    e = idx3.shape[1]
    ept = e // NS
    n_chunks = ept // K
    half = n_chunks // 2
    zrows = 128
    rpt = -(-n // (NS * zrows)) * zrows
    n_pad = NS * rpt
    ngroups = n_chunks // G
    assert e % NS == 0 and ept % K == 0 and n_chunks % G == 0 and ngroups % 2 == 0

    mesh = plsc.VectorSubcoreMesh(core_axis_name="c", subcore_axis_name="s")

    @functools.partial(
        pl.kernel,
        out_type=[
            jax.ShapeDtypeStruct((n_pad, NC * dh), jnp.float32),
            jax.ShapeDtypeStruct((NC, n_pad, LANES), jnp.float32),
        ],
        mesh=mesh,
        compiler_params=pltpu.CompilerParams(use_tc_tiling_on_sc=False),
        scratch_types=(
            [pltpu.VMEM((K,), jnp.int32) for _ in range(4 * G)]
            + [pltpu.VMEM((K, dh), jnp.float32) for _ in range(G)]
            + [
                pltpu.VMEM((K, LANES), jnp.float32),
                pltpu.VMEM((zrows, dh), jnp.float32),
                pltpu.VMEM((rpt, LANES), jnp.float32),
                pltpu.VMEM_SHARED((n_pad, dh), jnp.float32),
                pltpu.VMEM_SHARED((n_pad, LANES), jnp.float32),
            ]
            + [pltpu.SemaphoreType.DMA for _ in range(5 * G)]
        ),
    )
    def sc_kernel(h_hbm, idx_hbm, partial_hbm, deg_hbm, *scr):
        srcA = scr[:G]
        dstA = scr[G:2 * G]
        srcB = scr[2 * G:3 * G]
        dstB = scr[3 * G:4 * G]
        rows_v = scr[4 * G:5 * G]
        ones_v, zf_v, zd_v, acc_sh, deg_sh = scr[5 * G:5 * G + 5]
        sems = scr[5 * G + 5:]
        isemA, isemB, gsem, ssem, dsem = (
            sems[i * G:(i + 1) * G] for i in range(5))

        c = lax.axis_index("c")
        s = lax.axis_index("s")

        zero16 = jnp.zeros((LANES,), jnp.float32)
        one16 = jnp.ones((LANES,), jnp.float32)

        @pl.loop(0, zrows)
        def _(r):
            @pl.loop(0, dh // LANES)
            def _(q):
                zf_v[r, pl.ds(q * LANES, LANES)] = zero16

        @pl.loop(0, rpt)
        def _(r):
            zd_v[r, :] = zero16

        @pl.loop(0, K)
        def _(r):
            ones_v[r, :] = one16

        @pl.loop(0, rpt // zrows)
        def _(k):
            pltpu.sync_copy(zf_v, acc_sh.at[pl.ds(s * rpt + k * zrows, zrows)])

        pltpu.sync_copy(zd_v, deg_sh.at[pl.ds(s * rpt, rpt)])

        plsc.subcore_barrier()

        def cbase(g, b):
            return s * ept + (g * G + b) * K

        def issue_idx(g, b, sv, dv, sem):
            pltpu.async_copy(idx_hbm.at[c, pl.ds(cbase(g, b), K)], sv[b], sem[b])
            pltpu.async_copy(idx_hbm.at[2, pl.ds(cbase(g, b), K)], dv[b], sem[b])

        def wait_idx(g, b, sv, dv, sem):
            pltpu.make_async_copy(
                idx_hbm.at[c, pl.ds(cbase(g, b), K)], sv[b], sem[b]).wait()
            pltpu.make_async_copy(
                idx_hbm.at[2, pl.ds(cbase(g, b), K)], dv[b], sem[b]).wait()

        for b in range(G):
            issue_idx(0, b, srcA, dstA, isemA)
            issue_idx(1, b, srcB, dstB, isemB)
        for b in range(G):
            wait_idx(0, b, srcA, dstA, isemA)
            pltpu.async_copy(h_hbm.at[srcA[b]], rows_v[b], gsem[b])

        def subgroup(g, cur, nxt):
            (csrc, cdst, cisem), (nsrc, ndst, nisem) = cur, nxt
            scatters = []
            for b in range(G):
                pltpu.make_async_copy(
                    h_hbm.at[csrc[b]], rows_v[b], gsem[b]).wait()
                scatters.append(pltpu.async_copy(
                    rows_v[b], acc_sh.at[cdst[b]], ssem[b], add=True))
                i = g * G + b
                @pl.when(jnp.where(c == 0, i < half, i >= half))
                def _(b=b, i=i):
                    pltpu.async_copy(
                        ones_v, deg_sh.at[cdst[b]], dsem[b], add=True)

            for b in range(G):
                scatters[b].wait()
                i = g * G + b

                @pl.when(jnp.where(c == 0, i < half, i >= half))
                def _(b=b, i=i):
                    pltpu.make_async_copy(
                        ones_v, deg_sh.at[cdst[b]], dsem[b]).wait()

                @pl.when(g + 1 < ngroups)
                def _(b=b):
                    wait_idx(g + 1, b, nsrc, ndst, nisem)
                    pltpu.async_copy(h_hbm.at[nsrc[b]], rows_v[b], gsem[b])

                @pl.when(g + 2 < ngroups)
                def _(b=b):
                    issue_idx(g + 2, b, csrc, cdst, cisem)

        ringA = (srcA, dstA, isemA)
        ringB = (srcB, dstB, isemB)

        @pl.loop(0, ngroups // 2)
        def _(t):
            subgroup(2 * t, ringA, ringB)
            subgroup(2 * t + 1, ringB, ringA)

        plsc.subcore_barrier()

        pltpu.sync_copy(acc_sh.at[pl.ds(s * rpt, rpt)],
                        partial_hbm.at[pl.ds(s * rpt, rpt), pl.ds(c * dh, dh)])
        pltpu.sync_copy(deg_sh.at[pl.ds(s * rpt, rpt)],
                        deg_hbm.at[c, pl.ds(s * rpt, rpt)])

    return sc_kernel(hidden2, idx3)


def _mlp2(partial, degp, W_out, b_out, n, *, grid_n=2):
    d_hid = partial.shape[1]
    d_out = W_out.shape[1]
    blk = n // grid_n

    def body(p_ref, d_ref, w_ref, b_ref, o_ref):
        deg = d_ref[0][:, 0:1] + d_ref[1][:, 0:1]
        agg = p_ref[...] / jnp.maximum(deg, 1.0)
        o_ref[...] = (
            jnp.dot(agg, w_ref[...], preferred_element_type=jnp.float32)
            + b_ref[...]
        )

    return pl.pallas_call(
        body,
        grid=(grid_n,),
        in_specs=[
            pl.BlockSpec((blk, d_hid), lambda i: (i, 0)),
            pl.BlockSpec((NC, blk, LANES), lambda i: (0, i, 0)),
            pl.BlockSpec((d_hid, d_out), lambda i: (0, 0)),
            pl.BlockSpec((1, d_out), lambda i: (0, 0)),
        ],
        out_specs=pl.BlockSpec((blk, d_out), lambda i: (i, 0)),
        out_shape=jax.ShapeDtypeStruct((n, d_out), jnp.float32),
    )(partial, degp, W_out, b_out.reshape(1, d_out))


def kernel(node_features, edge_index, W1, b1, W_out, b_out):
    n, d_hid = node_features.shape[0], W1.shape[1]
    ei = edge_index.astype(jnp.int32)
    idx3 = (ei[jnp.array([0, 0, 1])] * jnp.array([[2], [2], [1]], jnp.int32)
            + jnp.array([[0], [1], [0]], jnp.int32))
    hidden = _mlp1(node_features, W1, b1)
    hidden2 = hidden.reshape(NC * n, d_hid // NC)
    partial, degp = _sc_aggregate(hidden2, idx3)
    return _mlp2(partial, degp, W_out, b_out, n)


# speedup vs baseline: 1.0351x; 1.0351x over previous
import functools

import jax
import jax.numpy as jnp
from jax import lax
from jax.experimental import pallas as pl
from jax.experimental.pallas import tpu as pltpu
from jax.experimental.pallas import tpu_sc as plsc

NC = 2
NS = 16
LANES = 16


def _mlp1(x, W1, b1, *, grid_n=2):
    n, d_in = x.shape
    d_hid = W1.shape[1]
    blk = n // grid_n

    def body(x_ref, w_ref, b_ref, o_ref):
        h = jnp.dot(x_ref[...], w_ref[...], preferred_element_type=jnp.float32)
        o_ref[...] = jax.nn.sigmoid(jnp.tanh(h + b_ref[...]))

    return pl.pallas_call(
        body,
        grid=(grid_n,),
        in_specs=[
            pl.BlockSpec((blk, d_in), lambda i: (i, 0)),
            pl.BlockSpec((d_in, d_hid), lambda i: (0, 0)),
            pl.BlockSpec((1, d_hid), lambda i: (0, 0)),
        ],
        out_specs=pl.BlockSpec((blk, d_hid), lambda i: (i, 0)),
        out_shape=jax.ShapeDtypeStruct((n, d_hid), jnp.float32),
    )(x, W1, b1.reshape(1, d_hid))


K = 80
G = 5


def _sc_aggregate(hidden2, idx3):
    n2, dh = hidden2.shape
    n = n2 // NC
    e = idx3.shape[1]
    ept = e // NS
    n_chunks = ept // K
    half = n_chunks // 2
    zrows = 128
    rpt = -(-n // (NS * zrows)) * zrows
    n_pad = NS * rpt
    ngroups = n_chunks // G
    assert e % NS == 0 and ept % K == 0 and n_chunks % G == 0 and ngroups % 2 == 0

    mesh = plsc.VectorSubcoreMesh(core_axis_name="c", subcore_axis_name="s")

    @functools.partial(
        pl.kernel,
        out_type=[
            jax.ShapeDtypeStruct((n_pad, NC * dh), jnp.float32),
            jax.ShapeDtypeStruct((NC, n_pad, LANES), jnp.float32),
        ],
        mesh=mesh,
        compiler_params=pltpu.CompilerParams(use_tc_tiling_on_sc=False),
        scratch_types=(
            [pltpu.VMEM((K,), jnp.int32) for _ in range(4 * G)]
            + [pltpu.VMEM((K, dh), jnp.float32) for _ in range(G)]
            + [
                pltpu.VMEM((K, LANES), jnp.float32),
                pltpu.VMEM((zrows, dh), jnp.float32),
                pltpu.VMEM((rpt, LANES), jnp.float32),
                pltpu.VMEM_SHARED((n_pad, dh), jnp.float32),
                pltpu.VMEM_SHARED((n_pad, LANES), jnp.float32),
            ]
            + [pltpu.SemaphoreType.DMA for _ in range(5 * G)]
        ),
    )
    def sc_kernel(h_hbm, idx_hbm, partial_hbm, deg_hbm, *scr):
        srcA = scr[:G]
        dstA = scr[G:2 * G]
        srcB = scr[2 * G:3 * G]
        dstB = scr[3 * G:4 * G]
        rows_v = scr[4 * G:5 * G]
        ones_v, zf_v, zd_v, acc_sh, deg_sh = scr[5 * G:5 * G + 5]
        sems = scr[5 * G + 5:]
        isemA, isemB, gsem, ssem, dsem = (
            sems[i * G:(i + 1) * G] for i in range(5))

        c = lax.axis_index("c")
        s = lax.axis_index("s")

        zero16 = jnp.zeros((LANES,), jnp.float32)
        one16 = jnp.ones((LANES,), jnp.float32)

        @pl.loop(0, zrows)
        def _(r):
            @pl.loop(0, dh // LANES)
            def _(q):
                zf_v[r, pl.ds(q * LANES, LANES)] = zero16

        @pl.loop(0, rpt)
        def _(r):
            zd_v[r, :] = zero16

        @pl.loop(0, K)
        def _(r):
            ones_v[r, :] = one16

        @pl.loop(0, rpt // zrows)
        def _(k):
            pltpu.sync_copy(zf_v, acc_sh.at[pl.ds(s * rpt + k * zrows, zrows)])

        pltpu.sync_copy(zd_v, deg_sh.at[pl.ds(s * rpt, rpt)])

        plsc.subcore_barrier()

        def cbase(g, b):
            return s * ept + (g * G + b) * K

        def issue_idx(g, b, sv, dv, sem):
            pltpu.async_copy(idx_hbm.at[c, pl.ds(cbase(g, b), K)], sv[b], sem[b])
            pltpu.async_copy(idx_hbm.at[2, pl.ds(cbase(g, b), K)], dv[b], sem[b])

        def wait_idx(g, b, sv, dv, sem):
            pltpu.make_async_copy(
                idx_hbm.at[c, pl.ds(cbase(g, b), K)], sv[b], sem[b]).wait()
            pltpu.make_async_copy(
                idx_hbm.at[2, pl.ds(cbase(g, b), K)], dv[b], sem[b]).wait()

        for b in range(G):
            issue_idx(0, b, srcA, dstA, isemA)
            issue_idx(1, b, srcB, dstB, isemB)
        for b in range(G):
            wait_idx(0, b, srcA, dstA, isemA)
            pltpu.async_copy(h_hbm.at[srcA[b]], rows_v[b], gsem[b])

        def subgroup(g, cur, nxt):
            (csrc, cdst, cisem), (nsrc, ndst, nisem) = cur, nxt
            scatters = []
            for b in range(G):
                pltpu.make_async_copy(
                    h_hbm.at[csrc[b]], rows_v[b], gsem[b]).wait()
                scatters.append(pltpu.async_copy(
                    rows_v[b], acc_sh.at[cdst[b]], ssem[b], add=True))
                i = g * G + b
                @pl.when(jnp.where(c == 0, i < half, i >= half))
                def _(b=b, i=i):
                    pltpu.async_copy(
                        ones_v, deg_sh.at[cdst[b]], dsem[b], add=True)

            for b in range(G):
                scatters[b].wait()
                i = g * G + b

                @pl.when(jnp.where(c == 0, i < half, i >= half))
                def _(b=b, i=i):
                    pltpu.make_async_copy(
                        ones_v, deg_sh.at[cdst[b]], dsem[b]).wait()

                @pl.when(g + 1 < ngroups)
                def _(b=b):
                    wait_idx(g + 1, b, nsrc, ndst, nisem)
                    pltpu.async_copy(h_hbm.at[nsrc[b]], rows_v[b], gsem[b])

                @pl.when(g + 2 < ngroups)
                def _(b=b):
                    issue_idx(g + 2, b, csrc, cdst, cisem)

        ringA = (srcA, dstA, isemA)
        ringB = (srcB, dstB, isemB)

        @pl.loop(0, ngroups // 2)
        def _(t):
            subgroup(2 * t, ringA, ringB)
            subgroup(2 * t + 1, ringB, ringA)

        plsc.subcore_barrier()

        pltpu.sync_copy(acc_sh.at[pl.ds(s * rpt, rpt)],
                        partial_hbm.at[pl.ds(s * rpt, rpt), pl.ds(c * dh, dh)])
        pltpu.sync_copy(deg_sh.at[pl.ds(s * rpt, rpt)],
                        deg_hbm.at[c, pl.ds(s * rpt, rpt)])

    return sc_kernel(hidden2, idx3)


def _mlp2(partial, degp, W_out, b_out, n, *, grid_n=2):
    d_hid = partial.shape[1]
    d_out = W_out.shape[1]
    blk = n // grid_n

    def body(p_ref, d_ref, w_ref, b_ref, o_ref):
        deg = d_ref[0][:, 0:1] + d_ref[1][:, 0:1]
        agg = p_ref[...] / jnp.maximum(deg, 1.0)
        o_ref[...] = (
            jnp.dot(agg, w_ref[...], preferred_element_type=jnp.float32)
            + b_ref[...]
        )

    return pl.pallas_call(
        body,
        grid=(grid_n,),
        in_specs=[
            pl.BlockSpec((blk, d_hid), lambda i: (i, 0)),
            pl.BlockSpec((NC, blk, LANES), lambda i: (0, i, 0)),
            pl.BlockSpec((d_hid, d_out), lambda i: (0, 0)),
            pl.BlockSpec((1, d_out), lambda i: (0, 0)),
        ],
        out_specs=pl.BlockSpec((blk, d_out), lambda i: (i, 0)),
        out_shape=jax.ShapeDtypeStruct((n, d_out), jnp.float32),
    )(partial, degp, W_out, b_out.reshape(1, d_out))


def kernel(node_features, edge_index, W1, b1, W_out, b_out):
    n, d_hid = node_features.shape[0], W1.shape[1]
    ei = edge_index.astype(jnp.int32)
    idx3 = jnp.stack([2 * ei[0], 2 * ei[0] + 1, ei[1]])
    hidden = _mlp1(node_features, W1, b1)
    hidden2 = hidden.reshape(NC * n, d_hid // NC)
    partial, degp = _sc_aggregate(hidden2, idx3)
    return _mlp2(partial, degp, W_out, b_out, n)
